# Initial kernel scaffold; baseline (speedup 1.0000x reference)
import jax
import jax.numpy as jnp
from jax.experimental import pallas as pl


def kernel(embeddings, queue, ptr):
    raise NotImplementedError("write your pallas kernel here")


# speedup vs baseline: 1.9738x; 1.9738x over previous
import jax
import jax.numpy as jnp
from jax.experimental import pallas as pl
from jax.experimental.pallas import tpu as pltpu

BANK = 1_000_000
EMB = 32
BS = 16384
LANES = 128
PACK = LANES // EMB
BANK_P = BANK // PACK
BS_P = BS // PACK
BR = 2_000
GRID = BANK_P // BR
EPAD = BS_P + 2 * BR


def _body(ptr_ref, emb_ref, q_ref, out_ref):
    i = pl.program_id(0)
    s = i * BR
    p = ptr_ref[0]

    o = jax.lax.rem(s - p + BANK_P, BANK_P)
    b = jnp.where(o >= BANK_P - BR, o - BANK_P, o)
    b = jnp.clip(b, -BR, BS_P)
    emb_slice = emb_ref[pl.ds(b + BR, BR), :]

    j = jax.lax.broadcasted_iota(jnp.int32, (BR, 1), 0)
    d0 = o + j
    delta = jnp.where(d0 >= BANK_P, d0 - BANK_P, d0)
    take = delta < BS_P
    out_ref[:, :] = jnp.where(take, emb_slice, q_ref[:, :])


def kernel(embeddings, queue, ptr):
    emb_p = jnp.pad(embeddings.reshape(BS_P, LANES), ((BR, BR), (0, 0)))
    q = queue.reshape(BANK_P, LANES)
    p = (jnp.asarray(ptr, jnp.int32) // PACK) % BANK_P
    out = pl.pallas_call(
        _body,
        grid=(GRID,),
        in_specs=[
            pl.BlockSpec(memory_space=pltpu.SMEM),
            pl.BlockSpec((EPAD, LANES), lambda i: (0, 0)),
            pl.BlockSpec((BR, LANES), lambda i: (i, 0)),
        ],
        out_specs=pl.BlockSpec((BR, LANES), lambda i: (i, 0)),
        out_shape=jax.ShapeDtypeStruct((BANK_P, LANES), jnp.float32),
    )(p.reshape(1), emb_p, q)
    return out.reshape(BANK, EMB)


# speedup vs baseline: 2.3303x; 1.1807x over previous
import jax
import jax.numpy as jnp
from jax.experimental import pallas as pl
from jax.experimental.pallas import tpu as pltpu

BANK = 1_000_000
EMB = 32
BS = 16384
BR = 4_000
GRID = BANK // BR
EPAD = BS + 2 * BR


def _body(ptr_ref, emb_ref, q_ref, out_ref):
    i = pl.program_id(0)
    s = i * BR
    p = ptr_ref[0]

    o = jax.lax.rem(s - p + BANK, BANK)
    b = jnp.where(o >= BANK - BR, o - BANK, o)
    b = jnp.clip(b, -BR, BS)
    emb_slice = emb_ref[pl.ds(b + BR, BR), :]

    j = jax.lax.broadcasted_iota(jnp.int32, (BR, 1), 0)
    d0 = o + j
    delta = jnp.where(d0 >= BANK, d0 - BANK, d0)
    take = delta < BS
    out_ref[:, :] = jnp.where(take, emb_slice, q_ref[:, :])


def kernel(embeddings, queue, ptr):
    emb_p = jnp.pad(embeddings, ((BR, BR), (0, 0)))
    p = jax.lax.rem(jnp.asarray(ptr, jnp.int32), BANK)
    return pl.pallas_call(
        _body,
        grid=(GRID,),
        in_specs=[
            pl.BlockSpec(memory_space=pltpu.SMEM),
            pl.BlockSpec((EPAD, EMB), lambda i: (0, 0)),
            pl.BlockSpec((BR, EMB), lambda i: (i, 0)),
        ],
        out_specs=pl.BlockSpec((BR, EMB), lambda i: (i, 0)),
        out_shape=jax.ShapeDtypeStruct((BANK, EMB), jnp.float32),
    )(p.reshape(1), emb_p, queue)
